# Initial kernel scaffold; baseline (speedup 1.0000x reference)
import jax
import jax.numpy as jnp
from jax.experimental import pallas as pl


def kernel(boxes, scores):
    raise NotImplementedError("write your pallas kernel here")


# speedup vs baseline: 4.9299x; 4.9299x over previous
import jax
import jax.numpy as jnp
from jax import lax
from jax.experimental import pallas as pl
from jax.experimental.pallas import tpu as pltpu
from jax.experimental.pallas import tpu_sc as plsc

N = 20000
MAX_DET = 100
IOU_T = 0.5
NTILES = 16
PER_TILE = 1280
P = NTILES * PER_TILE
CHUNKS = PER_TILE // 16
NEG_INF = float("-inf")
BIG = 2**30
MAXRETRY = 8

_C_MAX, _C_IDX, _C_X, _C_Y, _C_X2, _C_Y2, _C_AREA, _C_TAG1 = range(8)
_C_W, _C_H, _C_S = 8, 9, 10
_C_TAG2 = 15


def _splat_i(v):
  return jnp.broadcast_to(jnp.asarray(v, jnp.int32), (16,))


def _nms_body(x_hbm, y_hbm, w_hbm, h_hbm, s_hbm, out_hbm, comm_hbm,
              xv, yv, x2v, y2v, areav, wv, hv, sv, alivev, outv,
              rowv, bufv):
  sid = lax.axis_index("s")
  base = sid * PER_TILE
  lane = lax.iota(jnp.int32, 16)

  pltpu.sync_copy(x_hbm.at[pl.ds(base, PER_TILE)], xv)
  pltpu.sync_copy(y_hbm.at[pl.ds(base, PER_TILE)], yv)
  pltpu.sync_copy(w_hbm.at[pl.ds(base, PER_TILE)], wv)
  pltpu.sync_copy(h_hbm.at[pl.ds(base, PER_TILE)], hv)
  pltpu.sync_copy(s_hbm.at[pl.ds(base, PER_TILE)], sv)
  pltpu.sync_copy(s_hbm.at[pl.ds(base, PER_TILE)], alivev)

  m = jnp.broadcast_to(jnp.float32(NEG_INF), (16,))
  midx = jnp.broadcast_to(BIG, (16,))
  for j in range(CHUNKS):
    ds = pl.ds(j * 16, 16)
    xc = xv[ds]
    yc = yv[ds]
    wc = wv[ds]
    hc = hv[ds]
    x2v[ds] = xc + wc
    y2v[ds] = yc + hc
    areav[ds] = wc * hc
    a = alivev[ds]
    idxj = lane + (j * 16)
    if j == 0:
      m, midx = a, idxj
    else:
      better = a > m
      m = jnp.where(better, a, m)
      midx = jnp.where(better, idxj, midx)
  midx = midx + base

  tagmask = (lane == _C_TAG1) | (lane == _C_TAG2)

  def body(i, carry):
    m, midx = carry
    gm_loc = jnp.max(m)
    lidx = jnp.min(jnp.where(m == gm_loc, midx, BIG))
    offv = _splat_i(lidx - base)
    gmv = jnp.broadcast_to(gm_loc, (16,))
    lidxf = jnp.broadcast_to(lidx.astype(jnp.float32), (16,))
    cx = plsc.load_gather(xv, [offv])
    cy = plsc.load_gather(yv, [offv])
    cx2 = plsc.load_gather(x2v, [offv])
    cy2 = plsc.load_gather(y2v, [offv])
    car = plsc.load_gather(areav, [offv])
    cw = plsc.load_gather(wv, [offv])
    ch = plsc.load_gather(hv, [offv])
    cs = plsc.load_gather(sv, [offv])
    tagv = jnp.broadcast_to((i + 1).astype(jnp.float32), (16,))
    row = jnp.where(lane == _C_MAX, gmv,
          jnp.where(lane == _C_IDX, lidxf,
          jnp.where(lane == _C_X, cx,
          jnp.where(lane == _C_Y, cy,
          jnp.where(lane == _C_X2, cx2,
          jnp.where(lane == _C_Y2, cy2,
          jnp.where(lane == _C_AREA, car,
          jnp.where(lane == _C_W, cw,
          jnp.where(lane == _C_H, ch,
          jnp.where(lane == _C_S, cs,
          jnp.where(tagmask, tagv,
                    jnp.float32(0.0))))))))))))
    rowv[...] = row
    pltpu.sync_copy(rowv, comm_hbm.at[sid])
    plsc.subcore_barrier()

    def fresh():
      acc = jnp.broadcast_to(True, (16,))
      for j in range(NTILES):
        acc = acc & jnp.where(tagmask, bufv[j, :] == tagv, True)
      return jnp.min(acc.astype(jnp.int32)) == 1

    pltpu.sync_copy(comm_hbm, bufv)

    def attempt(k, nret):
      del k
      stale = jnp.logical_not(fresh())
      @pl.when(stale)
      def _():
        pltpu.sync_copy(comm_hbm, bufv)
      return nret + stale.astype(jnp.int32)

    lax.fori_loop(0, MAXRETRY, attempt, jnp.int32(0))

    zeros = _splat_i(_C_MAX)
    ones = _splat_i(_C_IDX)
    best = bufv[0, :]
    bm = jnp.take(best, zeros)
    bi = jnp.take(best, ones)
    for j in range(1, NTILES):
      r = bufv[j, :]
      rm = jnp.take(r, zeros)
      ri = jnp.take(r, ones)
      better = (rm > bm) | ((rm == bm) & (ri < bi))
      best = jnp.where(better, r, best)
      bm = jnp.where(better, rm, bm)
      bi = jnp.where(better, ri, bi)
    wx = jnp.take(best, _splat_i(_C_X))
    wy = jnp.take(best, _splat_i(_C_Y))
    wx2 = jnp.take(best, _splat_i(_C_X2))
    wy2 = jnp.take(best, _splat_i(_C_Y2))
    war = jnp.take(best, _splat_i(_C_AREA))

    operm = jnp.where(lane == 0, _C_X,
            jnp.where(lane == 1, _C_Y,
            jnp.where(lane == 2, _C_W,
            jnp.where(lane == 3, _C_H,
            jnp.where(lane == 4, _C_S, _C_MAX)))))
    outv[i, :] = jnp.take(best, operm)

    m2 = jnp.broadcast_to(jnp.float32(NEG_INF), (16,))
    midx2 = jnp.broadcast_to(BIG, (16,))
    for j in range(CHUNKS):
      ds = pl.ds(j * 16, 16)
      ix1 = jnp.maximum(wx, xv[ds])
      iy1 = jnp.maximum(wy, yv[ds])
      ix2 = jnp.minimum(wx2, x2v[ds])
      iy2 = jnp.minimum(wy2, y2v[ds])
      ia = jnp.maximum(ix2 - ix1, 0.0) * jnp.maximum(iy2 - iy1, 0.0)
      iou = ia / (war + areav[ds] - ia)
      a = jnp.where(iou > IOU_T, jnp.float32(NEG_INF), alivev[ds])
      alivev[ds] = a
      idxj = lane + (j * 16)
      if j == 0:
        m2, midx2 = a, idxj
      else:
        better = a > m2
        m2 = jnp.where(better, a, m2)
        midx2 = jnp.where(better, idxj, midx2)
    plsc.subcore_barrier()
    return m2, midx2 + base

  lax.fori_loop(0, MAX_DET, body, (m, midx))

  @pl.when(sid == 0)
  def _():
    pltpu.sync_copy(outv, out_hbm)


@jax.jit
def kernel(boxes, scores):
  pad = P - N
  x = jnp.pad(boxes[:, 0], (0, pad))
  y = jnp.pad(boxes[:, 1], (0, pad))
  w = jnp.pad(boxes[:, 2], (0, pad))
  h = jnp.pad(boxes[:, 3], (0, pad))
  s = jnp.pad(scores, (0, pad), constant_values=NEG_INF)

  f32 = jnp.float32
  run = pl.kernel(
      _nms_body,
      out_type=(jax.ShapeDtypeStruct((MAX_DET, 16), f32),
                jax.ShapeDtypeStruct((NTILES, 16), f32)),
      mesh=plsc.VectorSubcoreMesh(
          core_axis_name="c", subcore_axis_name="s", num_cores=1),
      compiler_params=pltpu.CompilerParams(needs_layout_passes=False),
      scratch_types=[
          pltpu.VMEM((PER_TILE,), f32),
          pltpu.VMEM((PER_TILE,), f32),
          pltpu.VMEM((PER_TILE,), f32),
          pltpu.VMEM((PER_TILE,), f32),
          pltpu.VMEM((PER_TILE,), f32),
          pltpu.VMEM((PER_TILE,), f32),
          pltpu.VMEM((PER_TILE,), f32),
          pltpu.VMEM((PER_TILE,), f32),
          pltpu.VMEM((PER_TILE,), f32),
          pltpu.VMEM((MAX_DET, 16), f32),
          pltpu.VMEM((16,), f32),
          pltpu.VMEM((NTILES, 16), f32),
      ],
  )
  out, _ = run(x, y, w, h, s)
  return out[:, :5]


# speedup vs baseline: 5.3148x; 1.0781x over previous
import jax
import jax.numpy as jnp
from jax import lax
from jax.experimental import pallas as pl
from jax.experimental.pallas import tpu as pltpu
from jax.experimental.pallas import tpu_sc as plsc

N = 20000
MAX_DET = 100
IOU_T = 0.5
NTILES = 16
PER_TILE = 1280
P = NTILES * PER_TILE
CHUNKS = PER_TILE // 16
NEG_INF = float("-inf")
BIG = 2**30
MAXRETRY = 8

_C_MAX, _C_IDX, _C_X, _C_Y, _C_X2, _C_Y2, _C_AREA, _C_TAG1 = range(8)
_C_W, _C_H, _C_S = 8, 9, 10
_C_TAG2 = 15


def _splat_i(v):
  return jnp.broadcast_to(jnp.asarray(v, jnp.int32), (16,))


def _nms_body(x_hbm, y_hbm, w_hbm, h_hbm, s_hbm, out_hbm, comm_hbm,
              xv, yv, x2v, y2v, areav, wv, hv, sv, alivev, outv,
              rowv, bufv):
  sid = lax.axis_index("s")
  base = sid * PER_TILE
  lane = lax.iota(jnp.int32, 16)

  pltpu.sync_copy(x_hbm.at[pl.ds(base, PER_TILE)], xv)
  pltpu.sync_copy(y_hbm.at[pl.ds(base, PER_TILE)], yv)
  pltpu.sync_copy(w_hbm.at[pl.ds(base, PER_TILE)], wv)
  pltpu.sync_copy(h_hbm.at[pl.ds(base, PER_TILE)], hv)
  pltpu.sync_copy(s_hbm.at[pl.ds(base, PER_TILE)], sv)
  pltpu.sync_copy(s_hbm.at[pl.ds(base, PER_TILE)], alivev)

  m = jnp.broadcast_to(jnp.float32(NEG_INF), (16,))
  midx = jnp.broadcast_to(BIG, (16,))
  for j in range(CHUNKS):
    ds = pl.ds(j * 16, 16)
    xc = xv[ds]
    yc = yv[ds]
    wc = wv[ds]
    hc = hv[ds]
    x2v[ds] = xc + wc
    y2v[ds] = yc + hc
    areav[ds] = wc * hc
    a = alivev[ds]
    idxj = lane + (j * 16)
    if j == 0:
      m, midx = a, idxj
    else:
      better = a > m
      m = jnp.where(better, a, m)
      midx = jnp.where(better, idxj, midx)
  midx = midx + base

  tagmask = (lane == _C_TAG1) | (lane == _C_TAG2)

  def body(i, carry):
    m, midx = carry
    gm_loc = jnp.max(m)
    lidx = jnp.min(jnp.where(m == gm_loc, midx, BIG))
    offv = _splat_i(lidx - base)
    gmv = jnp.broadcast_to(gm_loc, (16,))
    lidxf = jnp.broadcast_to(lidx.astype(jnp.float32), (16,))
    cx = plsc.load_gather(xv, [offv])
    cy = plsc.load_gather(yv, [offv])
    cx2 = plsc.load_gather(x2v, [offv])
    cy2 = plsc.load_gather(y2v, [offv])
    car = plsc.load_gather(areav, [offv])
    cw = plsc.load_gather(wv, [offv])
    ch = plsc.load_gather(hv, [offv])
    cs = plsc.load_gather(sv, [offv])
    tagv = jnp.broadcast_to((i + 1).astype(jnp.float32), (16,))
    row = jnp.where(lane == _C_MAX, gmv,
          jnp.where(lane == _C_IDX, lidxf,
          jnp.where(lane == _C_X, cx,
          jnp.where(lane == _C_Y, cy,
          jnp.where(lane == _C_X2, cx2,
          jnp.where(lane == _C_Y2, cy2,
          jnp.where(lane == _C_AREA, car,
          jnp.where(lane == _C_W, cw,
          jnp.where(lane == _C_H, ch,
          jnp.where(lane == _C_S, cs,
          jnp.where(tagmask, tagv,
                    jnp.float32(0.0))))))))))))
    rowv[...] = row
    pltpu.sync_copy(rowv, comm_hbm.at[sid])
    plsc.subcore_barrier()

    def fresh():
      acc = jnp.broadcast_to(True, (16,))
      for j in range(NTILES):
        acc = acc & jnp.where(tagmask, bufv[j, :] == tagv, True)
      return jnp.min(acc.astype(jnp.int32)) == 1

    pltpu.sync_copy(comm_hbm, bufv)

    def _stale_cond(st):
      k, stale = st
      return stale & (k < MAXRETRY)

    def _retry(st):
      k, _ = st
      pltpu.sync_copy(comm_hbm, bufv)
      return k + 1, jnp.logical_not(fresh())

    lax.while_loop(_stale_cond, _retry,
                   (jnp.int32(0), jnp.logical_not(fresh())))

    zeros = _splat_i(_C_MAX)
    ones = _splat_i(_C_IDX)
    cand = []
    for j in range(NTILES):
      r = bufv[j, :]
      cand.append((r, jnp.take(r, zeros), jnp.take(r, ones)))
    while len(cand) > 1:
      nxt = []
      for a, b in zip(cand[0::2], cand[1::2]):
        (ra, ma, ia), (rb, mb, ib) = a, b
        better = (mb > ma) | ((mb == ma) & (ib < ia))
        nxt.append((jnp.where(better, rb, ra),
                    jnp.where(better, mb, ma),
                    jnp.where(better, ib, ia)))
      cand = nxt
    best = cand[0][0]
    wx = jnp.take(best, _splat_i(_C_X))
    wy = jnp.take(best, _splat_i(_C_Y))
    wx2 = jnp.take(best, _splat_i(_C_X2))
    wy2 = jnp.take(best, _splat_i(_C_Y2))
    war = jnp.take(best, _splat_i(_C_AREA))

    operm = jnp.where(lane == 0, _C_X,
            jnp.where(lane == 1, _C_Y,
            jnp.where(lane == 2, _C_W,
            jnp.where(lane == 3, _C_H,
            jnp.where(lane == 4, _C_S, _C_MAX)))))
    outv[i, :] = jnp.take(best, operm)

    m2 = jnp.broadcast_to(jnp.float32(NEG_INF), (16,))
    midx2 = jnp.broadcast_to(BIG, (16,))
    for j in range(CHUNKS):
      ds = pl.ds(j * 16, 16)
      ix1 = jnp.maximum(wx, xv[ds])
      iy1 = jnp.maximum(wy, yv[ds])
      ix2 = jnp.minimum(wx2, x2v[ds])
      iy2 = jnp.minimum(wy2, y2v[ds])
      ia = jnp.maximum(ix2 - ix1, 0.0) * jnp.maximum(iy2 - iy1, 0.0)
      iou = ia / (war + areav[ds] - ia)
      a = jnp.where(iou > IOU_T, jnp.float32(NEG_INF), alivev[ds])
      alivev[ds] = a
      idxj = lane + (j * 16)
      if j == 0:
        m2, midx2 = a, idxj
      else:
        better = a > m2
        m2 = jnp.where(better, a, m2)
        midx2 = jnp.where(better, idxj, midx2)
    plsc.subcore_barrier()
    return m2, midx2 + base

  lax.fori_loop(0, MAX_DET, body, (m, midx))

  @pl.when(sid == 0)
  def _():
    pltpu.sync_copy(outv, out_hbm)


@jax.jit
def kernel(boxes, scores):
  pad = P - N
  x = jnp.pad(boxes[:, 0], (0, pad))
  y = jnp.pad(boxes[:, 1], (0, pad))
  w = jnp.pad(boxes[:, 2], (0, pad))
  h = jnp.pad(boxes[:, 3], (0, pad))
  s = jnp.pad(scores, (0, pad), constant_values=NEG_INF)

  f32 = jnp.float32
  run = pl.kernel(
      _nms_body,
      out_type=(jax.ShapeDtypeStruct((MAX_DET, 16), f32),
                jax.ShapeDtypeStruct((NTILES, 16), f32)),
      mesh=plsc.VectorSubcoreMesh(
          core_axis_name="c", subcore_axis_name="s", num_cores=1),
      compiler_params=pltpu.CompilerParams(needs_layout_passes=False),
      scratch_types=[
          pltpu.VMEM((PER_TILE,), f32),
          pltpu.VMEM((PER_TILE,), f32),
          pltpu.VMEM((PER_TILE,), f32),
          pltpu.VMEM((PER_TILE,), f32),
          pltpu.VMEM((PER_TILE,), f32),
          pltpu.VMEM((PER_TILE,), f32),
          pltpu.VMEM((PER_TILE,), f32),
          pltpu.VMEM((PER_TILE,), f32),
          pltpu.VMEM((PER_TILE,), f32),
          pltpu.VMEM((MAX_DET, 16), f32),
          pltpu.VMEM((16,), f32),
          pltpu.VMEM((NTILES, 16), f32),
      ],
  )
  out, _ = run(x, y, w, h, s)
  return out[:, :5]
